# baseline (speedup 1.0000x reference)
import functools

import jax
import jax.numpy as jnp
from jax import lax
from jax.experimental import pallas as pl
from jax.experimental.pallas import tpu as pltpu
from jax.experimental.pallas import tpu_sc as plsc

_TOPK = 8
_E = 64
_M_BLK = 1024
_NC, _NS, _L = 2, 16, 16
_NW = _NC * _NS


def _gating_block(x_ref, w_ref, b_ref, scores_ref):
    logits_t = jax.lax.dot_general(
        w_ref[...], x_ref[...], (((1,), (1,)), ((), ())),
        preferred_element_type=jnp.float32)
    logits_t = logits_t + b_ref[...][:, 0:1]
    m = jnp.max(logits_t, axis=0, keepdims=True)
    e = jnp.exp(logits_t - m)
    s = jnp.sum(e, axis=0, keepdims=True)
    scores_ref[...] = (e / s).T


def _tc_scores(x, W, b2, row0, n_rows):
    d_model = x.shape[1]
    n_exp = W.shape[0]
    blk0 = row0 // _M_BLK
    return pl.pallas_call(
        _gating_block,
        grid=(n_rows // _M_BLK,),
        in_specs=[
            pl.BlockSpec((_M_BLK, d_model), lambda i: (blk0 + i, 0)),
            pl.BlockSpec((n_exp, d_model), lambda i: (0, 0)),
            pl.BlockSpec((n_exp, 128), lambda i: (0, 0)),
        ],
        out_specs=pl.BlockSpec((_M_BLK, n_exp), lambda i: (i, 0)),
        out_shape=jax.ShapeDtypeStruct((n_rows, n_exp), jnp.float32),
    )(x, W, b2)


def _sc_topk_call(scores, rpw):
    n_tokens = scores.shape[0]
    mesh = plsc.VectorSubcoreMesh(core_axis_name="c", subcore_axis_name="s")

    @functools.partial(
        pl.kernel, mesh=mesh,
        compiler_params=pltpu.CompilerParams(needs_layout_passes=False, use_tc_tiling_on_sc=False),
        out_type=[
            jax.ShapeDtypeStruct((_NW, rpw * _TOPK), jnp.float32),
            jax.ShapeDtypeStruct((_NW, rpw * _TOPK), jnp.int32),
        ],
        scratch_types=[
            pltpu.VMEM((rpw, _E), jnp.float32),
            pltpu.VMEM((rpw * _TOPK + 16,), jnp.float32),
            pltpu.VMEM((rpw * _TOPK + 16,), jnp.int32),
        ],
    )
    def sc_topk(scores_hbm, vals_hbm, idx_hbm, slab, vals_v, idx_v):
        wid = lax.axis_index("s") * _NC + lax.axis_index("c")
        base = wid * rpw
        pltpu.sync_copy(scores_hbm.at[pl.ds(base, rpw)], slab)

        iota = lax.iota(jnp.int32, _L)
        lo8 = iota < _TOPK
        pm_rev8 = jnp.maximum(7 - iota, 0)
        pm_shift8 = jnp.maximum(iota - 8, 0)

        def perm(v, p):
            return lax.gather(
                v, p[:, None],
                lax.GatherDimensionNumbers(
                    offset_dims=(), collapsed_slice_dims=(0,),
                    start_index_map=(0,)),
                (1,), mode=lax.GatherScatterMode.PROMISE_IN_BOUNDS)

        def set_merge(av, ai, bv, bi):
            wv = perm(bv, pm_rev8)
            wi = perm(bi, pm_rev8)
            keep = av >= wv
            return jnp.where(keep, av, wv), jnp.where(keep, ai, wi)

        @plsc.parallel_loop(0, rpw, unroll=4)
        def row(r):
            sv, si = [], []
            for c in range(4):
                v = slab[r, pl.ds(16 * c, 16)]
                kv, ki = plsc.sort_key_val(v, iota + 16 * c, descending=True)
                sv.append(kv)
                si.append(ki)
            t01v, t01i = set_merge(sv[0], si[0], sv[1], si[1])
            t23v, t23i = set_merge(sv[2], si[2], sv[3], si[3])
            cv = jnp.where(lo8, t01v, perm(t23v, pm_shift8))
            ci = jnp.where(lo8, t01i, perm(t23i, pm_shift8))
            fv, fi = plsc.sort_key_val(cv, ci, descending=True)
            o = pl.multiple_of(r * _TOPK, _TOPK)
            plsc.store_compressed(vals_v.at[pl.ds(o, 16)], fv, mask=lo8)
            plsc.store_compressed(idx_v.at[pl.ds(o, 16)], fi, mask=lo8)
        pltpu.sync_copy(vals_v.at[pl.ds(0, rpw * _TOPK)], vals_hbm.at[wid])
        pltpu.sync_copy(idx_v.at[pl.ds(0, rpw * _TOPK)], idx_hbm.at[wid])

    return sc_topk(scores)


def kernel(x, W, b):
    n_tokens = x.shape[0]
    n_exp = W.shape[0]
    b2 = jnp.broadcast_to(b.reshape(n_exp, 1), (n_exp, 128))
    rpw = n_tokens // _NW
    scores = _tc_scores(x, W, b2, 0, n_tokens)
    vals_w, idx_w = _sc_topk_call(scores, rpw)
    vals = vals_w.reshape(n_tokens, _TOPK)
    idx = idx_w.reshape(n_tokens, _TOPK)
    return (vals, idx, scores)


# speedup vs baseline: 1.0008x; 1.0008x over previous
import functools

import jax
import jax.numpy as jnp
from jax import lax
from jax.experimental import pallas as pl
from jax.experimental.pallas import tpu as pltpu
from jax.experimental.pallas import tpu_sc as plsc

_TOPK = 8
_E = 64
_M_BLK = 1024
_NC, _NS, _L = 2, 16, 16
_NW = _NC * _NS


def _gating_block(x_ref, w_ref, b_ref, scores_ref):
    logits_t = jax.lax.dot_general(
        w_ref[...], x_ref[...], (((1,), (1,)), ((), ())),
        preferred_element_type=jnp.float32)
    logits_t = logits_t + b_ref[...][:, 0:1]
    m = jnp.max(logits_t, axis=0, keepdims=True)
    e = jnp.exp(logits_t - m)
    s = jnp.sum(e, axis=0, keepdims=True)
    scores_ref[...] = (e / s).T


def _tc_scores(x, W, b2, row0, n_rows):
    d_model = x.shape[1]
    n_exp = W.shape[0]
    blk0 = row0 // _M_BLK
    return pl.pallas_call(
        _gating_block,
        grid=(n_rows // _M_BLK,),
        in_specs=[
            pl.BlockSpec((_M_BLK, d_model), lambda i: (blk0 + i, 0)),
            pl.BlockSpec((n_exp, d_model), lambda i: (0, 0)),
            pl.BlockSpec((n_exp, 128), lambda i: (0, 0)),
        ],
        out_specs=pl.BlockSpec((_M_BLK, n_exp), lambda i: (i, 0)),
        out_shape=jax.ShapeDtypeStruct((n_rows, n_exp), jnp.float32),
    )(x, W, b2)


def _sc_topk_call(scores_w, rpw):
    mesh = plsc.VectorSubcoreMesh(core_axis_name="c", subcore_axis_name="s")

    @functools.partial(
        pl.kernel, mesh=mesh,
        compiler_params=pltpu.CompilerParams(needs_layout_passes=False, use_tc_tiling_on_sc=False),
        out_type=[
            jax.ShapeDtypeStruct((_NW, rpw * _TOPK), jnp.float32),
            jax.ShapeDtypeStruct((_NW, rpw * _TOPK), jnp.int32),
        ],
        scratch_types=[
            pltpu.VMEM((rpw * _E,), jnp.float32),
            pltpu.VMEM((rpw * _TOPK + 16,), jnp.float32),
            pltpu.VMEM((rpw * _TOPK + 16,), jnp.int32),
        ],
    )
    def sc_topk(scores_hbm, vals_hbm, idx_hbm, slab, vals_v, idx_v):
        wid = lax.axis_index("s") * _NC + lax.axis_index("c")
        pltpu.sync_copy(scores_hbm.at[wid], slab)

        iota = lax.iota(jnp.int32, _L)
        lo8 = iota < _TOPK
        pm_rev8 = jnp.maximum(7 - iota, 0)
        pm_shift8 = jnp.maximum(iota - 8, 0)

        def perm(v, p):
            return lax.gather(
                v, p[:, None],
                lax.GatherDimensionNumbers(
                    offset_dims=(), collapsed_slice_dims=(0,),
                    start_index_map=(0,)),
                (1,), mode=lax.GatherScatterMode.PROMISE_IN_BOUNDS)

        def set_merge(av, ai, bv, bi):
            wv = perm(bv, pm_rev8)
            wi = perm(bi, pm_rev8)
            keep = av >= wv
            return jnp.where(keep, av, wv), jnp.where(keep, ai, wi)

        @plsc.parallel_loop(0, rpw, unroll=4)
        def row(r):
            off = pl.multiple_of(r * _E, _E)
            sv, si = [], []
            for c in range(4):
                v = slab[pl.ds(off + 16 * c, 16)]
                kv, ki = plsc.sort_key_val(v, iota + 16 * c, descending=True)
                sv.append(kv)
                si.append(ki)
            t01v, t01i = set_merge(sv[0], si[0], sv[1], si[1])
            t23v, t23i = set_merge(sv[2], si[2], sv[3], si[3])
            cv = jnp.where(lo8, t01v, perm(t23v, pm_shift8))
            ci = jnp.where(lo8, t01i, perm(t23i, pm_shift8))
            fv, fi = plsc.sort_key_val(cv, ci, descending=True)
            o = pl.multiple_of(r * _TOPK, _TOPK)
            plsc.store_compressed(vals_v.at[pl.ds(o, 16)], fv, mask=lo8)
            plsc.store_compressed(idx_v.at[pl.ds(o, 16)], fi, mask=lo8)
        pltpu.sync_copy(vals_v.at[pl.ds(0, rpw * _TOPK)], vals_hbm.at[wid])
        pltpu.sync_copy(idx_v.at[pl.ds(0, rpw * _TOPK)], idx_hbm.at[wid])

    return sc_topk(scores_w)


def kernel(x, W, b):
    n_tokens = x.shape[0]
    n_exp = W.shape[0]
    b2 = jnp.broadcast_to(b.reshape(n_exp, 1), (n_exp, 128))
    rpw = n_tokens // _NW
    scores = _tc_scores(x, W, b2, 0, n_tokens)
    vals_w, idx_w = _sc_topk_call(scores.reshape(_NW, rpw * _E), rpw)
    vals = vals_w.reshape(n_tokens, _TOPK)
    idx = idx_w.reshape(n_tokens, _TOPK)
    return (vals, idx, scores)


# speedup vs baseline: 1.0123x; 1.0115x over previous
import functools

import jax
import jax.numpy as jnp
from jax import lax
from jax.experimental import pallas as pl
from jax.experimental.pallas import tpu as pltpu
from jax.experimental.pallas import tpu_sc as plsc

_TOPK = 8
_E = 64
_M_BLK = 1024
_NC, _NS, _L = 2, 16, 16
_NW = _NC * _NS


def _gating_block(x_ref, w_ref, b_ref, scores_ref):
    logits_t = jax.lax.dot_general(
        w_ref[...], x_ref[...], (((1,), (1,)), ((), ())),
        preferred_element_type=jnp.float32)
    logits_t = logits_t + b_ref[...][:, 0:1]
    m = jnp.max(logits_t, axis=0, keepdims=True)
    e = jnp.exp(logits_t - m)
    s = jnp.sum(e, axis=0, keepdims=True)
    scores_ref[...] = (e / s).T


def _tc_scores(x, W, b2, row0, n_rows):
    d_model = x.shape[1]
    n_exp = W.shape[0]
    blk0 = row0 // _M_BLK
    return pl.pallas_call(
        _gating_block,
        grid=(n_rows // _M_BLK,),
        in_specs=[
            pl.BlockSpec((_M_BLK, d_model), lambda i: (blk0 + i, 0)),
            pl.BlockSpec((n_exp, d_model), lambda i: (0, 0)),
            pl.BlockSpec((n_exp, 128), lambda i: (0, 0)),
        ],
        out_specs=pl.BlockSpec((_M_BLK, n_exp), lambda i: (i, 0)),
        out_shape=jax.ShapeDtypeStruct((n_rows, n_exp), jnp.float32),
    )(x, W, b2)


def _sc_topk_call(scores_w, rpw):
    mesh = plsc.VectorSubcoreMesh(core_axis_name="c", subcore_axis_name="s")

    @functools.partial(
        pl.kernel, mesh=mesh,
        compiler_params=pltpu.CompilerParams(needs_layout_passes=False),
        out_type=[
            jax.ShapeDtypeStruct((_NW, rpw * _TOPK), jnp.float32),
            jax.ShapeDtypeStruct((_NW, rpw * _TOPK), jnp.int32),
        ],
        scratch_types=[
            pltpu.VMEM((rpw * _E,), jnp.float32),
            pltpu.VMEM((rpw * _TOPK + 16,), jnp.float32),
            pltpu.VMEM((rpw * _TOPK + 16,), jnp.int32),
        ],
    )
    def sc_topk(scores_hbm, vals_hbm, idx_hbm, slab, vals_v, idx_v):
        wid = lax.axis_index("s") * _NC + lax.axis_index("c")
        pltpu.sync_copy(scores_hbm.at[wid], slab)

        iota = lax.iota(jnp.int32, _L)
        lo8 = iota < _TOPK
        pm_rev8 = jnp.maximum(7 - iota, 0)
        pm_rev8h = jnp.minimum(23 - iota, 15)

        def perm(v, p):
            return lax.gather(
                v, p[:, None],
                lax.GatherDimensionNumbers(
                    offset_dims=(), collapsed_slice_dims=(0,),
                    start_index_map=(0,)),
                (1,), mode=lax.GatherScatterMode.PROMISE_IN_BOUNDS)

        def set_merge(av, ai, bv, bi, pm):
            wv = perm(bv, pm)
            wi = perm(bi, pm)
            keep = av >= wv
            return jnp.where(keep, av, wv), jnp.where(keep, ai, wi)

        @plsc.parallel_loop(0, rpw, unroll=4)
        def row(r):
            off = pl.multiple_of(r * _E, _E)
            sv, si = [], []
            for c in range(4):
                v = slab[pl.ds(off + 16 * c, 16)]
                kv, ki = plsc.sort_key_val(v, iota + 16 * c,
                                           descending=(c < 2))
                sv.append(kv)
                si.append(ki)
            t01v, t01i = set_merge(sv[0], si[0], sv[1], si[1], pm_rev8)
            t23v, t23i = set_merge(sv[2], si[2], sv[3], si[3], pm_rev8h)
            cv = jnp.where(lo8, t01v, t23v)
            ci = jnp.where(lo8, t01i, t23i)
            fv, fi = plsc.sort_key_val(cv, ci, descending=True)
            o = pl.multiple_of(r * _TOPK, _TOPK)
            plsc.store_compressed(vals_v.at[pl.ds(o, 16)], fv, mask=lo8)
            plsc.store_compressed(idx_v.at[pl.ds(o, 16)], fi, mask=lo8)
        pltpu.sync_copy(vals_v.at[pl.ds(0, rpw * _TOPK)], vals_hbm.at[wid])
        pltpu.sync_copy(idx_v.at[pl.ds(0, rpw * _TOPK)], idx_hbm.at[wid])

    return sc_topk(scores_w)


def kernel(x, W, b):
    n_tokens = x.shape[0]
    n_exp = W.shape[0]
    b2 = jnp.broadcast_to(b.reshape(n_exp, 1), (n_exp, 128))
    rpw = n_tokens // _NW
    scores = _tc_scores(x, W, b2, 0, n_tokens)
    vals_w, idx_w = _sc_topk_call(scores.reshape(_NW, rpw * _E), rpw)
    vals = vals_w.reshape(n_tokens, _TOPK)
    idx = idx_w.reshape(n_tokens, _TOPK)
    return (vals, idx, scores)
